# baseline (speedup 1.0000x reference)
import jax
import jax.numpy as jnp
from jax.experimental import pallas as pl
from jax.experimental.pallas import tpu as pltpu


def _body(h_ref, w1a_ref, w1b_ref, b1_ref, w2_ref, b2_ref, w3_ref, b3_ref,
          o_ref):
    h = h_ref[0]
    T = h.shape[0]
    ri = jax.lax.broadcasted_iota(jnp.int32, (T, T), 0)
    ci = jax.lax.broadcasted_iota(jnp.int32, (T, T), 1)
    rf = jnp.maximum(ri, 1).astype(jnp.float32)
    coeff = jnp.where(ci < ri, 1.0 / rf, 0.0)
    pm = jnp.dot(coeff, h, preferred_element_type=jnp.float32)
    z1 = (jnp.dot(pm, w1a_ref[...], preferred_element_type=jnp.float32)
          + jnp.dot(h, w1b_ref[...], preferred_element_type=jnp.float32)
          + b1_ref[...])
    h1 = jnp.maximum(z1, 0.0)
    z2 = jnp.dot(h1, w2_ref[...], preferred_element_type=jnp.float32) + b2_ref[...]
    h2 = jnp.maximum(z2, 0.0)
    row = jnp.sum(h2 * w3_ref[...], axis=1) + b3_ref[0, 0]
    o_ref[...] = row.reshape(1, 1, T)


def _mlp_call(hidden_state, W1, b1, W2, b2, W3, b3):
    B, T, D = hidden_state.shape
    H2 = W1.shape[1]
    H = W2.shape[1]
    out = pl.pallas_call(
        _body,
        grid=(B,),
        in_specs=[
            pl.BlockSpec((1, T, D), lambda b: (b, 0, 0)),
            pl.BlockSpec((D, H2), lambda b: (0, 0)),
            pl.BlockSpec((D, H2), lambda b: (1, 0)),
            pl.BlockSpec((1, H2), lambda b: (0, 0)),
            pl.BlockSpec((H2, H), lambda b: (0, 0)),
            pl.BlockSpec((1, H), lambda b: (0, 0)),
            pl.BlockSpec((1, H), lambda b: (0, 0)),
            pl.BlockSpec((1, 1), lambda b: (0, 0)),
        ],
        out_specs=pl.BlockSpec((1, 1, T), lambda b: (b, 0, 0)),
        out_shape=jax.ShapeDtypeStruct((B, 1, T), jnp.float32),
        compiler_params=pltpu.CompilerParams(
            dimension_semantics=("arbitrary",),
            vmem_limit_bytes=64 * 1024 * 1024,
        ),
        name="ffd_fused",
    )(hidden_state, W1, W1, b1.reshape(1, H2), W2, b2.reshape(1, H),
      W3.reshape(1, H), b3.reshape(1, 1))
    return out.reshape(B, T, 1)


def kernel(hidden_state, W1, b1, W2, b2, W3, b3):
    B = hidden_state.shape[0]
    devs = jax.devices()
    n = 2 if (len(devs) >= 2 and B % 2 == 0) else 1
    if n == 1:
        return _mlp_call(hidden_state, W1, b1, W2, b2, W3, b3)
    mesh = jax.sharding.Mesh(devs[:n], ("c",))
    P = jax.sharding.PartitionSpec
    f = jax.shard_map(
        _mlp_call, mesh=mesh,
        in_specs=(P("c"), P(), P(), P(), P(), P(), P()),
        out_specs=P("c"), check_vma=False)
    return f(hidden_state, W1, b1, W2, b2, W3, b3)


# speedup vs baseline: 1.5150x; 1.5150x over previous
import jax
import jax.numpy as jnp
from jax.experimental import pallas as pl
from jax.experimental.pallas import tpu as pltpu


def _body(h_ref, w1a_ref, w1b_ref, b1_ref, w2_ref, b2_ref, w3_ref, b3_ref,
          o_ref):
    h = h_ref[0]
    T = h.shape[0]
    ri = jax.lax.broadcasted_iota(jnp.int32, (T, T), 0)
    ci = jax.lax.broadcasted_iota(jnp.int32, (T, T), 1)
    rf = jnp.maximum(ri, 1).astype(jnp.float32)
    coeff = jnp.where(ci < ri, 1.0 / rf, 0.0)
    pm = jnp.dot(coeff, h, preferred_element_type=jnp.float32)
    z1 = (jnp.dot(pm.astype(jnp.bfloat16), w1a_ref[...],
                  preferred_element_type=jnp.float32)
          + jnp.dot(h.astype(jnp.bfloat16), w1b_ref[...],
                    preferred_element_type=jnp.float32)
          + b1_ref[...])
    h1 = jnp.maximum(z1, 0.0).astype(jnp.bfloat16)
    z2 = jnp.dot(h1, w2_ref[...], preferred_element_type=jnp.float32) + b2_ref[...]
    h2 = jnp.maximum(z2, 0.0)
    row = jnp.sum(h2 * w3_ref[...], axis=1) + b3_ref[0, 0]
    o_ref[...] = row.reshape(1, 1, T)


def _mlp_call(hidden_state, W1, b1, W2, b2, W3, b3):
    B, T, D = hidden_state.shape
    H2 = W1.shape[1]
    H = W2.shape[1]
    W1b16 = W1.astype(jnp.bfloat16)
    out = pl.pallas_call(
        _body,
        grid=(B,),
        in_specs=[
            pl.BlockSpec((1, T, D), lambda b: (b, 0, 0)),
            pl.BlockSpec((D, H2), lambda b: (0, 0)),
            pl.BlockSpec((D, H2), lambda b: (1, 0)),
            pl.BlockSpec((1, H2), lambda b: (0, 0)),
            pl.BlockSpec((H2, H), lambda b: (0, 0)),
            pl.BlockSpec((1, H), lambda b: (0, 0)),
            pl.BlockSpec((1, H), lambda b: (0, 0)),
            pl.BlockSpec((1, 1), lambda b: (0, 0)),
        ],
        out_specs=pl.BlockSpec((1, 1, T), lambda b: (b, 0, 0)),
        out_shape=jax.ShapeDtypeStruct((B, 1, T), jnp.float32),
        compiler_params=pltpu.CompilerParams(
            dimension_semantics=("arbitrary",),
            vmem_limit_bytes=64 * 1024 * 1024,
        ),
        name="ffd_fused",
    )(hidden_state, W1b16, W1b16, b1.reshape(1, H2), W2.astype(jnp.bfloat16),
      b2.reshape(1, H), W3.reshape(1, H), b3.reshape(1, 1))
    return out.reshape(B, T, 1)


def kernel(hidden_state, W1, b1, W2, b2, W3, b3):
    return _mlp_call(hidden_state, W1, b1, W2, b2, W3, b3)


# speedup vs baseline: 1.5778x; 1.0414x over previous
import jax
import jax.numpy as jnp
from jax.experimental import pallas as pl
from jax.experimental.pallas import tpu as pltpu

_GB = 2


def _one_batch(h, w1a_ref, w1b_ref, b1_ref, w2_ref, b2_ref, w3_ref, b3_ref):
    T = h.shape[0]
    ri = jax.lax.broadcasted_iota(jnp.int32, (T, T), 0)
    ci = jax.lax.broadcasted_iota(jnp.int32, (T, T), 1)
    rf = jnp.maximum(ri, 1).astype(jnp.float32)
    coeff = jnp.where(ci < ri, 1.0 / rf, 0.0)
    pm = jnp.dot(coeff, h, preferred_element_type=jnp.float32)
    z1 = (jnp.dot(pm.astype(jnp.bfloat16), w1a_ref[...],
                  preferred_element_type=jnp.float32)
          + jnp.dot(h.astype(jnp.bfloat16), w1b_ref[...],
                    preferred_element_type=jnp.float32)
          + b1_ref[...])
    h1 = jnp.maximum(z1, 0.0).astype(jnp.bfloat16)
    z2 = jnp.dot(h1, w2_ref[...], preferred_element_type=jnp.float32) + b2_ref[...]
    h2 = jnp.maximum(z2, 0.0)
    row = jnp.sum(h2 * w3_ref[...], axis=1) + b3_ref[0, 0]
    return row


def _body(h_ref, w1a_ref, w1b_ref, b1_ref, w2_ref, b2_ref, w3_ref, b3_ref,
          o_ref):
    T = h_ref.shape[1]
    for g in range(_GB):
        row = _one_batch(h_ref[g], w1a_ref, w1b_ref, b1_ref, w2_ref, b2_ref,
                         w3_ref, b3_ref)
        o_ref[g] = row.reshape(1, T)


def _mlp_call(hidden_state, W1, b1, W2, b2, W3, b3):
    B, T, D = hidden_state.shape
    H2 = W1.shape[1]
    H = W2.shape[1]
    W1b16 = W1.astype(jnp.bfloat16)
    out = pl.pallas_call(
        _body,
        grid=(B // _GB,),
        in_specs=[
            pl.BlockSpec((_GB, T, D), lambda b: (b, 0, 0)),
            pl.BlockSpec((D, H2), lambda b: (0, 0)),
            pl.BlockSpec((D, H2), lambda b: (1, 0)),
            pl.BlockSpec((1, H2), lambda b: (0, 0)),
            pl.BlockSpec((H2, H), lambda b: (0, 0)),
            pl.BlockSpec((1, H), lambda b: (0, 0)),
            pl.BlockSpec((1, H), lambda b: (0, 0)),
            pl.BlockSpec((1, 1), lambda b: (0, 0)),
        ],
        out_specs=pl.BlockSpec((_GB, 1, T), lambda b: (b, 0, 0)),
        out_shape=jax.ShapeDtypeStruct((B, 1, T), jnp.float32),
        compiler_params=pltpu.CompilerParams(
            dimension_semantics=("arbitrary",),
            vmem_limit_bytes=64 * 1024 * 1024,
        ),
        name="ffd_fused",
    )(hidden_state, W1b16, W1b16, b1.reshape(1, H2), W2.astype(jnp.bfloat16),
      b2.reshape(1, H), W3.reshape(1, H), b3.reshape(1, 1))
    return out.reshape(B, T, 1)


def kernel(hidden_state, W1, b1, W2, b2, W3, b3):
    return _mlp_call(hidden_state, W1, b1, W2, b2, W3, b3)


# speedup vs baseline: 1.8892x; 1.1974x over previous
import jax
import jax.numpy as jnp
from jax.experimental import pallas as pl
from jax.experimental.pallas import tpu as pltpu

_TC = 8


def _body(x_ref, w1a_ref, w1b_ref, b1_ref, w2_ref, b2_ref, w3_ref, b3_ref,
          o_ref, acc_ref, pm_ref):
    step = pl.program_id(0)
    R, D = x_ref.shape
    B = R // _TC

    @pl.when(step == 0)
    def _():
        acc_ref[...] = jnp.zeros_like(acc_ref)

    s = acc_ref[...]
    for j in range(_TC):
        t = step * _TC + j
        inv = 1.0 / jnp.maximum(jnp.float32(t), 1.0)
        pm_ref[j * B:(j + 1) * B, :] = (s * inv).astype(jnp.bfloat16)
        s = s + x_ref[j * B:(j + 1) * B, :]
    acc_ref[...] = s

    x16 = x_ref[...].astype(jnp.bfloat16)
    z1 = (jnp.dot(x16, w1b_ref[...], preferred_element_type=jnp.float32)
          + jnp.dot(pm_ref[...], w1a_ref[...],
                    preferred_element_type=jnp.float32)
          + b1_ref[...])
    h1 = jnp.maximum(z1, 0.0).astype(jnp.bfloat16)
    z2 = jnp.dot(h1, w2_ref[...], preferred_element_type=jnp.float32) + b2_ref[...]
    h2 = jnp.maximum(z2, 0.0)
    row = jnp.sum(h2 * w3_ref[...], axis=1) + b3_ref[0, 0]
    o_ref[...] = row.reshape(1, 1, R)


def _mlp_call(hidden_state, W1, b1, W2, b2, W3, b3):
    B, T, D = hidden_state.shape
    H2 = W1.shape[1]
    H = W2.shape[1]
    R = _TC * B
    steps = (T * B + R - 1) // R
    W1b16 = W1.astype(jnp.bfloat16)
    x = hidden_state.transpose(1, 0, 2).reshape(T * B, D)
    out = pl.pallas_call(
        _body,
        grid=(steps,),
        in_specs=[
            pl.BlockSpec((R, D), lambda i: (i, 0)),
            pl.BlockSpec((D, H2), lambda i: (0, 0)),
            pl.BlockSpec((D, H2), lambda i: (1, 0)),
            pl.BlockSpec((1, H2), lambda i: (0, 0)),
            pl.BlockSpec((H2, H), lambda i: (0, 0)),
            pl.BlockSpec((1, H), lambda i: (0, 0)),
            pl.BlockSpec((1, H), lambda i: (0, 0)),
            pl.BlockSpec((1, 1), lambda i: (0, 0)),
        ],
        out_specs=pl.BlockSpec((1, 1, R), lambda i: (i, 0, 0)),
        out_shape=jax.ShapeDtypeStruct((steps, 1, R), jnp.float32),
        scratch_shapes=[
            pltpu.VMEM((B, D), jnp.float32),
            pltpu.VMEM((R, D), jnp.bfloat16),
        ],
        compiler_params=pltpu.CompilerParams(
            dimension_semantics=("arbitrary",),
            vmem_limit_bytes=56 * 1024 * 1024,
        ),
        name="ffd_fused",
    )(x, W1b16, W1b16, b1.reshape(1, H2), W2.astype(jnp.bfloat16),
      b2.reshape(1, H), W3.reshape(1, H), b3.reshape(1, 1))
    return out.reshape(steps * R)[:T * B].reshape(T, B).transpose(1, 0)[:, :, None]


def kernel(hidden_state, W1, b1, W2, b2, W3, b3):
    return _mlp_call(hidden_state, W1, b1, W2, b2, W3, b3)


# speedup vs baseline: 2.0004x; 1.0588x over previous
import jax
import jax.numpy as jnp
from jax.experimental import pallas as pl
from jax.experimental.pallas import tpu as pltpu

_TC = 8


def _body(x_ref, w1a_ref, w1b_ref, b1_ref, w2_ref, b2_ref, w3_ref, b3_ref,
          o_ref, acc_ref, pm_ref):
    step = pl.program_id(0)
    R, D = x_ref.shape
    B = R // _TC

    @pl.when(step == 0)
    def _():
        acc_ref[...] = jnp.zeros_like(acc_ref)

    s = acc_ref[...]
    for j in range(_TC):
        t = step * _TC + j
        inv = 1.0 / jnp.maximum(jnp.float32(t), 1.0)
        pm_ref[j * B:(j + 1) * B, :] = (s * inv).astype(jnp.bfloat16)
        s = s + x_ref[j * B:(j + 1) * B, :]
    acc_ref[...] = s

    CH = 4
    rpc = R // CH
    for c in range(CH):
        sl = slice(c * rpc, (c + 1) * rpc)
        xc = x_ref[sl, :].astype(jnp.bfloat16)
        z1 = (jnp.dot(xc, w1b_ref[...], preferred_element_type=jnp.float32)
              + jnp.dot(pm_ref[sl, :], w1a_ref[...],
                        preferred_element_type=jnp.float32)
              + b1_ref[...])
        h1 = jnp.maximum(z1, 0.0).astype(jnp.bfloat16)
        z2 = (jnp.dot(h1, w2_ref[...], preferred_element_type=jnp.float32)
              + b2_ref[...])
        h2 = jnp.maximum(z2, 0.0)
        row = jnp.sum(h2 * w3_ref[...], axis=1) + b3_ref[0, 0]
        o_ref[0, 0, sl] = row


def _mlp_call(hidden_state, W1, b1, W2, b2, W3, b3):
    B, T, D = hidden_state.shape
    H2 = W1.shape[1]
    H = W2.shape[1]
    R = _TC * B
    steps = (T * B + R - 1) // R
    W1b16 = W1.astype(jnp.bfloat16)
    x = hidden_state.transpose(1, 0, 2).reshape(T * B, D)
    out = pl.pallas_call(
        _body,
        grid=(steps,),
        in_specs=[
            pl.BlockSpec((R, D), lambda i: (i, 0)),
            pl.BlockSpec((D, H2), lambda i: (0, 0)),
            pl.BlockSpec((D, H2), lambda i: (1, 0)),
            pl.BlockSpec((1, H2), lambda i: (0, 0)),
            pl.BlockSpec((H2, H), lambda i: (0, 0)),
            pl.BlockSpec((1, H), lambda i: (0, 0)),
            pl.BlockSpec((1, H), lambda i: (0, 0)),
            pl.BlockSpec((1, 1), lambda i: (0, 0)),
        ],
        out_specs=pl.BlockSpec((1, 1, R), lambda i: (i, 0, 0)),
        out_shape=jax.ShapeDtypeStruct((steps, 1, R), jnp.float32),
        scratch_shapes=[
            pltpu.VMEM((B, D), jnp.float32),
            pltpu.VMEM((R, D), jnp.bfloat16),
        ],
        compiler_params=pltpu.CompilerParams(
            dimension_semantics=("arbitrary",),
            vmem_limit_bytes=56 * 1024 * 1024,
        ),
        name="ffd_fused",
    )(x, W1b16, W1b16, b1.reshape(1, H2), W2.astype(jnp.bfloat16),
      b2.reshape(1, H), W3.reshape(1, H), b3.reshape(1, 1))
    return out.reshape(steps * R)[:T * B].reshape(T, B).transpose(1, 0)[:, :, None]


def kernel(hidden_state, W1, b1, W2, b2, W3, b3):
    return _mlp_call(hidden_state, W1, b1, W2, b2, W3, b3)


# speedup vs baseline: 2.0104x; 1.0050x over previous
import jax
import jax.numpy as jnp
from jax.experimental import pallas as pl
from jax.experimental.pallas import tpu as pltpu

_TC = 16


def _body(x_ref, w1_ref, b1_ref, w2_ref, b2_ref, w3_ref, b3_ref,
          o_ref, acc_ref, ft_ref):
    step = pl.program_id(0)
    R, D = x_ref.shape
    B = R // _TC

    @pl.when(step == 0)
    def _():
        acc_ref[...] = jnp.zeros_like(acc_ref)

    s = acc_ref[...]
    for j in range(_TC):
        t = step * _TC + j
        inv = 1.0 / jnp.maximum(jnp.float32(t), 1.0)
        xj = x_ref[j * B:(j + 1) * B, :]
        ft_ref[j * B:(j + 1) * B, :D] = (s * inv).astype(jnp.bfloat16)
        ft_ref[j * B:(j + 1) * B, D:] = xj.astype(jnp.bfloat16)
        s = s + xj
    acc_ref[...] = s

    rpc = 256
    for c in range(R // rpc):
        sl = slice(c * rpc, (c + 1) * rpc)
        z1 = (jnp.dot(ft_ref[sl, :], w1_ref[...],
                      preferred_element_type=jnp.float32)
              + b1_ref[...])
        h1 = jnp.maximum(z1, 0.0).astype(jnp.bfloat16)
        z2 = (jnp.dot(h1, w2_ref[...], preferred_element_type=jnp.float32)
              + b2_ref[...])
        h2 = jnp.maximum(z2, 0.0)
        row = jnp.sum(h2 * w3_ref[...], axis=1) + b3_ref[0, 0]
        o_ref[0, 0, sl] = row


def _mlp_call(hidden_state, W1, b1, W2, b2, W3, b3):
    B, T, D = hidden_state.shape
    H2 = W1.shape[1]
    H = W2.shape[1]
    R = _TC * B
    steps = (T * B + R - 1) // R
    x = hidden_state.transpose(1, 0, 2).reshape(T * B, D)
    out = pl.pallas_call(
        _body,
        grid=(steps,),
        in_specs=[
            pl.BlockSpec((R, D), lambda i: (i, 0)),
            pl.BlockSpec((2 * D, H2), lambda i: (0, 0)),
            pl.BlockSpec((1, H2), lambda i: (0, 0)),
            pl.BlockSpec((H2, H), lambda i: (0, 0)),
            pl.BlockSpec((1, H), lambda i: (0, 0)),
            pl.BlockSpec((1, H), lambda i: (0, 0)),
            pl.BlockSpec((1, 1), lambda i: (0, 0)),
        ],
        out_specs=pl.BlockSpec((1, 1, R), lambda i: (i, 0, 0)),
        out_shape=jax.ShapeDtypeStruct((steps, 1, R), jnp.float32),
        scratch_shapes=[
            pltpu.VMEM((B, D), jnp.float32),
            pltpu.VMEM((R, 2 * D), jnp.bfloat16),
        ],
        compiler_params=pltpu.CompilerParams(
            dimension_semantics=("arbitrary",),
            vmem_limit_bytes=56 * 1024 * 1024,
        ),
        name="ffd_fused",
    )(x, W1.astype(jnp.bfloat16), b1.reshape(1, H2), W2.astype(jnp.bfloat16),
      b2.reshape(1, H), W3.reshape(1, H), b3.reshape(1, 1))
    return out.reshape(steps * R)[:T * B].reshape(T, B).transpose(1, 0)[:, :, None]


def kernel(hidden_state, W1, b1, W2, b2, W3, b3):
    return _mlp_call(hidden_state, W1, b1, W2, b2, W3, b3)


# speedup vs baseline: 2.0148x; 1.0022x over previous
import jax
import jax.numpy as jnp
from jax.experimental import pallas as pl
from jax.experimental.pallas import tpu as pltpu

_TC = 16


def _body(x_ref, w1_ref, b1_ref, w2_ref, b2_ref, w3_ref, b3_ref,
          o_ref, acc_ref, ft_ref):
    step = pl.program_id(0)
    R, D = x_ref.shape
    B = R // _TC

    @pl.when(step == 0)
    def _():
        acc_ref[...] = jnp.zeros_like(acc_ref)

    s = acc_ref[...]
    for j in range(_TC):
        t = step * _TC + j
        inv = 1.0 / jnp.maximum(jnp.float32(t), 1.0)
        xj = x_ref[j * B:(j + 1) * B, :]
        ft_ref[j * B:(j + 1) * B, :D] = (s * inv).astype(jnp.bfloat16)
        ft_ref[j * B:(j + 1) * B, D:] = xj.astype(jnp.bfloat16)
        s = s + xj
    acc_ref[...] = s

    rpc = 128
    for c in range(R // rpc):
        sl = slice(c * rpc, (c + 1) * rpc)
        z1 = (jnp.dot(ft_ref[sl, :], w1_ref[...],
                      preferred_element_type=jnp.float32)
              + b1_ref[...])
        h1 = jnp.maximum(z1, 0.0).astype(jnp.bfloat16)
        z2 = (jnp.dot(h1, w2_ref[...], preferred_element_type=jnp.float32)
              + b2_ref[...])
        h2 = jnp.maximum(z2, 0.0)
        row = jnp.sum(h2 * w3_ref[...], axis=1) + b3_ref[0, 0]
        o_ref[0, 0, sl] = row


def _mlp_call(hidden_state, W1, b1, W2, b2, W3, b3):
    B, T, D = hidden_state.shape
    H2 = W1.shape[1]
    H = W2.shape[1]
    R = _TC * B
    steps = (T * B + R - 1) // R
    x = hidden_state.transpose(1, 0, 2).reshape(T * B, D)
    out = pl.pallas_call(
        _body,
        grid=(steps,),
        in_specs=[
            pl.BlockSpec((R, D), lambda i: (i, 0)),
            pl.BlockSpec((2 * D, H2), lambda i: (0, 0)),
            pl.BlockSpec((1, H2), lambda i: (0, 0)),
            pl.BlockSpec((H2, H), lambda i: (0, 0)),
            pl.BlockSpec((1, H), lambda i: (0, 0)),
            pl.BlockSpec((1, H), lambda i: (0, 0)),
            pl.BlockSpec((1, 1), lambda i: (0, 0)),
        ],
        out_specs=pl.BlockSpec((1, 1, R), lambda i: (i, 0, 0)),
        out_shape=jax.ShapeDtypeStruct((steps, 1, R), jnp.float32),
        scratch_shapes=[
            pltpu.VMEM((B, D), jnp.float32),
            pltpu.VMEM((R, 2 * D), jnp.bfloat16),
        ],
        compiler_params=pltpu.CompilerParams(
            dimension_semantics=("arbitrary",),
            vmem_limit_bytes=56 * 1024 * 1024,
        ),
        name="ffd_fused",
    )(x, W1.astype(jnp.bfloat16), b1.reshape(1, H2), W2.astype(jnp.bfloat16),
      b2.reshape(1, H), W3.reshape(1, H), b3.reshape(1, 1))
    return out.reshape(steps * R)[:T * B].reshape(T, B).transpose(1, 0)[:, :, None]


def kernel(hidden_state, W1, b1, W2, b2, W3, b3):
    return _mlp_call(hidden_state, W1, b1, W2, b2, W3, b3)
